# baseline (device time: 14285 ns/iter reference)
import jax
import jax.numpy as jnp
from jax import lax
from jax.experimental import pallas as pl
from jax.experimental.pallas import tpu as pltpu

N_DEV = 4
EPS = 1e-5


def kernel(x, gamma, beta):
    m, n_loc = x.shape
    n_glob = n_loc * N_DEV

    def body(x_ref, gamma_ref, beta_ref, out_ref, stats_ref, send_sems, recv_sems):
        my_pos = lax.axis_index("i")

        xf = x_ref[...].astype(jnp.float32)
        psum = jnp.sum(xf, axis=1, keepdims=True)
        psq = jnp.sum(xf * xf, axis=1, keepdims=True)
        stats_ref[my_pos] = jnp.concatenate([psum, psq], axis=1)

        barrier_sem = pltpu.get_barrier_semaphore()
        for off in range(1, N_DEV):
            peer = lax.rem(my_pos + off, N_DEV)
            pl.semaphore_signal(
                barrier_sem, inc=1,
                device_id=(peer,), device_id_type=pl.DeviceIdType.MESH,
            )
        pl.semaphore_wait(barrier_sem, N_DEV - 1)

        sends = []
        for off in range(1, N_DEV):
            peer = lax.rem(my_pos + off, N_DEV)
            rdma = pltpu.make_async_remote_copy(
                src_ref=stats_ref.at[my_pos],
                dst_ref=stats_ref.at[my_pos],
                send_sem=send_sems.at[off - 1],
                recv_sem=recv_sems.at[my_pos],
                device_id=(peer,),
                device_id_type=pl.DeviceIdType.MESH,
            )
            rdma.start()
            sends.append(rdma)

        for off in range(1, N_DEV):
            peer = lax.rem(my_pos + off, N_DEV)
            recv = pltpu.make_async_remote_copy(
                src_ref=stats_ref.at[peer],
                dst_ref=stats_ref.at[peer],
                send_sem=send_sems.at[off - 1],
                recv_sem=recv_sems.at[peer],
                device_id=(peer,),
                device_id_type=pl.DeviceIdType.MESH,
            )
            recv.wait_recv()

        tot = (stats_ref[0] + stats_ref[1]) + (stats_ref[2] + stats_ref[3])
        mean = tot[:, 0:1] / n_glob
        var = tot[:, 1:2] / n_glob - mean * mean
        inv = lax.rsqrt(var + EPS)
        g = gamma_ref[...].astype(jnp.float32)[None, :]
        b = beta_ref[...].astype(jnp.float32)[None, :]
        out_ref[...] = (g * ((xf - mean) * inv) + b).astype(out_ref.dtype)

        for rdma in sends:
            rdma.wait_send()

    return pl.pallas_call(
        body,
        out_shape=jax.ShapeDtypeStruct((m, n_loc), x.dtype),
        in_specs=[
            pl.BlockSpec(memory_space=pltpu.VMEM),
            pl.BlockSpec(memory_space=pltpu.VMEM),
            pl.BlockSpec(memory_space=pltpu.VMEM),
        ],
        out_specs=pl.BlockSpec(memory_space=pltpu.VMEM),
        scratch_shapes=[
            pltpu.VMEM((N_DEV, m, 2), jnp.float32),
            pltpu.SemaphoreType.DMA((N_DEV - 1,)),
            pltpu.SemaphoreType.DMA((N_DEV,)),
        ],
        compiler_params=pltpu.CompilerParams(collective_id=0),
    )(x, gamma, beta)


# device time: 14146 ns/iter; 1.0098x vs baseline; 1.0098x over previous
import jax
import jax.numpy as jnp
from jax import lax
from jax.experimental import pallas as pl
from jax.experimental.pallas import tpu as pltpu

N_DEV = 4
EPS = 1e-5


def kernel(x, gamma, beta):
    m, n_loc = x.shape
    n_glob = n_loc * N_DEV

    def body(x_ref, gamma_ref, beta_ref, out_ref, stats_ref, send_sems, recv_sems):
        my_pos = lax.axis_index("i")

        barrier_sem = pltpu.get_barrier_semaphore()
        for off in range(1, N_DEV):
            peer = lax.rem(my_pos + off, N_DEV)
            pl.semaphore_signal(
                barrier_sem, inc=1,
                device_id=(peer,), device_id_type=pl.DeviceIdType.MESH,
            )

        xf = x_ref[...].astype(jnp.float32)
        psum = jnp.sum(xf, axis=1, keepdims=True)
        psq = jnp.sum(xf * xf, axis=1, keepdims=True)
        stats_ref[my_pos] = jnp.concatenate([psum, psq], axis=1)

        pl.semaphore_wait(barrier_sem, N_DEV - 1)

        sends = []
        for off in range(1, N_DEV):
            peer = lax.rem(my_pos + off, N_DEV)
            rdma = pltpu.make_async_remote_copy(
                src_ref=stats_ref.at[my_pos],
                dst_ref=stats_ref.at[my_pos],
                send_sem=send_sems.at[off - 1],
                recv_sem=recv_sems.at[my_pos],
                device_id=(peer,),
                device_id_type=pl.DeviceIdType.MESH,
            )
            rdma.start()
            sends.append(rdma)

        g = gamma_ref[...].astype(jnp.float32)[None, :]
        b = beta_ref[...].astype(jnp.float32)[None, :]
        xg = xf * g

        for off in range(1, N_DEV):
            peer = lax.rem(my_pos + off, N_DEV)
            recv = pltpu.make_async_remote_copy(
                src_ref=stats_ref.at[peer],
                dst_ref=stats_ref.at[peer],
                send_sem=send_sems.at[off - 1],
                recv_sem=recv_sems.at[peer],
                device_id=(peer,),
                device_id_type=pl.DeviceIdType.MESH,
            )
            recv.wait_recv()

        tot = (stats_ref[0] + stats_ref[1]) + (stats_ref[2] + stats_ref[3])
        mean = tot[:, 0:1] / n_glob
        var = tot[:, 1:2] / n_glob - mean * mean
        inv = lax.rsqrt(var + EPS)
        out_ref[...] = (inv * xg - (inv * mean) * g + b).astype(out_ref.dtype)

        for rdma in sends:
            rdma.wait_send()

    return pl.pallas_call(
        body,
        out_shape=jax.ShapeDtypeStruct((m, n_loc), x.dtype),
        in_specs=[
            pl.BlockSpec(memory_space=pltpu.VMEM),
            pl.BlockSpec(memory_space=pltpu.VMEM),
            pl.BlockSpec(memory_space=pltpu.VMEM),
        ],
        out_specs=pl.BlockSpec(memory_space=pltpu.VMEM),
        scratch_shapes=[
            pltpu.VMEM((N_DEV, m, 2), jnp.float32),
            pltpu.SemaphoreType.DMA((N_DEV - 1,)),
            pltpu.SemaphoreType.DMA((N_DEV,)),
        ],
        compiler_params=pltpu.CompilerParams(collective_id=0),
    )(x, gamma, beta)


# device time: 14057 ns/iter; 1.0162x vs baseline; 1.0063x over previous
import jax
import jax.numpy as jnp
from jax import lax
from jax.experimental import pallas as pl
from jax.experimental.pallas import tpu as pltpu

N_DEV = 4
EPS = 1e-5


def kernel(x, gamma, beta):
    m, n_loc = x.shape
    n_glob = n_loc * N_DEV

    def body(x_ref, gamma_ref, beta_ref, out_ref, stats_ref, send_sems, recv_sems):
        my_pos = lax.axis_index("i")

        barrier_sem = pltpu.get_barrier_semaphore()
        for off in range(1, N_DEV):
            peer = lax.rem(my_pos + off, N_DEV)
            pl.semaphore_signal(
                barrier_sem, inc=1,
                device_id=(peer,), device_id_type=pl.DeviceIdType.MESH,
            )

        xf = x_ref[...].astype(jnp.float32)
        psum = jnp.sum(xf, axis=1, keepdims=True)
        psq = jnp.sum(xf * xf, axis=1, keepdims=True)
        stats_ref[my_pos] = jnp.concatenate([psum, psq], axis=1)

        pl.semaphore_wait(barrier_sem, N_DEV - 1)

        sends = []
        for off in (2, 1, 3):
            peer = lax.rem(my_pos + off, N_DEV)
            rdma = pltpu.make_async_remote_copy(
                src_ref=stats_ref.at[my_pos],
                dst_ref=stats_ref.at[my_pos],
                send_sem=send_sems.at[off - 1],
                recv_sem=recv_sems.at[my_pos],
                device_id=(peer,),
                device_id_type=pl.DeviceIdType.MESH,
            )
            rdma.start()
            sends.append(rdma)

        g = gamma_ref[...].astype(jnp.float32)[None, :]
        b = beta_ref[...].astype(jnp.float32)[None, :]
        xg = xf * g

        for off in range(1, N_DEV):
            peer = lax.rem(my_pos + off, N_DEV)
            recv = pltpu.make_async_remote_copy(
                src_ref=stats_ref.at[peer],
                dst_ref=stats_ref.at[peer],
                send_sem=send_sems.at[off - 1],
                recv_sem=recv_sems.at[peer],
                device_id=(peer,),
                device_id_type=pl.DeviceIdType.MESH,
            )
            recv.wait_recv()

        tot = (stats_ref[0] + stats_ref[1]) + (stats_ref[2] + stats_ref[3])
        mean = tot[:, 0:1] / n_glob
        var = tot[:, 1:2] / n_glob - mean * mean
        inv = lax.rsqrt(var + EPS)
        out_ref[...] = (inv * xg + (b - (inv * mean) * g)).astype(out_ref.dtype)

        for rdma in sends:
            rdma.wait_send()

    return pl.pallas_call(
        body,
        out_shape=jax.ShapeDtypeStruct((m, n_loc), jnp.bfloat16),
        in_specs=[
            pl.BlockSpec(memory_space=pltpu.VMEM),
            pl.BlockSpec(memory_space=pltpu.VMEM),
            pl.BlockSpec(memory_space=pltpu.VMEM),
        ],
        out_specs=pl.BlockSpec(memory_space=pltpu.VMEM),
        scratch_shapes=[
            pltpu.VMEM((N_DEV, m, 2), jnp.float32),
            pltpu.SemaphoreType.DMA((N_DEV - 1,)),
            pltpu.SemaphoreType.DMA((N_DEV,)),
        ],
        compiler_params=pltpu.CompilerParams(collective_id=0),
    )(x, gamma, beta)
